# baseline (speedup 1.0000x reference)
import functools

import jax
import jax.numpy as jnp
from jax import lax
from jax.experimental import pallas as pl
from jax.experimental.pallas import tpu as pltpu
from jax.experimental.pallas import tpu_sc as plsc

P = 131072
D = 64
L = 16
NW = 32
PW = 2 * P // NW
CH = 128
NCH = PW // CH
NBUF = 4
G = CH // L

f32 = jnp.float32
i32 = jnp.int32


def _rsqrt(x):
    i = lax.bitcast_convert_type(x, i32)
    y = lax.bitcast_convert_type(jnp.int32(0x5F3759DF) - jnp.right_shift(i, 1), f32)
    for _ in range(3):
        y = y * (f32(1.5) - f32(0.5) * x * y * y)
    return y


def _sc_body(emb, a1, po, a2, ne, scl, out,
             aidx_v, bidx_v,
             ar0, ar1, ar2, ar3, br0, br1, br2, br3,
             scl_v, stage_v, m_ref, s_ref, t_ref,
             sem0, sem1, sem2, sem3):
    c = lax.axis_index("c")
    s = lax.axis_index("s")
    wid = s * 2 + c
    lane = lax.iota(i32, L)
    half = wid < (NW // 2)
    w2 = jnp.where(half, wid, wid - NW // 2)
    posf = jnp.where(half, f32(1.0), f32(0.0))
    base = w2 * PW

    pltpu.sync_copy(scl, scl_v)

    @pl.when(half)
    def _():
        pltpu.sync_copy(a1.at[pl.ds(base, PW)], aidx_v)
        pltpu.sync_copy(po.at[pl.ds(base, PW)], bidx_v)

    @pl.when(jnp.logical_not(half))
    def _():
        pltpu.sync_copy(a2.at[pl.ds(base, PW)], aidx_v)
        pltpu.sync_copy(ne.at[pl.ds(base, PW)], bidx_v)

    m_ref[...] = jnp.full((L,), -3.0e38, f32)
    s_ref[...] = jnp.zeros((L,), f32)
    t_ref[...] = jnp.zeros((L,), f32)

    arows = [ar0, ar1, ar2, ar3]
    brows = [br0, br1, br2, br3]
    sems = [sem0, sem1, sem2, sem3]

    def issue(j, b):
        pltpu.async_copy(emb.at[aidx_v.at[pl.ds(j * CH, CH)]], arows[b], sems[b])
        pltpu.async_copy(emb.at[bidx_v.at[pl.ds(j * CH, CH)]], brows[b], sems[b])

    def drain(b):
        pltpu.make_async_copy(emb.at[aidx_v.at[pl.ds(0, CH)]], arows[b], sems[b]).wait()
        pltpu.make_async_copy(emb.at[bidx_v.at[pl.ds(0, CH)]], brows[b], sems[b]).wait()

    def compute(b):
        a_r, b_r = arows[b], brows[b]
        scale = scl_v[...]

        def grp(g, _):
            dacc = jnp.zeros((L,), f32)
            naacc = jnp.zeros((L,), f32)
            nbacc = jnp.zeros((L,), f32)
            for i in range(L):
                p = g * L + i
                dot_v = jnp.zeros((L,), f32)
                na_v = jnp.zeros((L,), f32)
                nb_v = jnp.zeros((L,), f32)
                for k in range(D // L):
                    av = a_r[p, pl.ds(k * L, L)]
                    bv = b_r[p, pl.ds(k * L, L)]
                    dot_v = dot_v + av * bv
                    na_v = na_v + av * av
                    nb_v = nb_v + bv * bv
                sel = lane == i
                dacc = jnp.where(sel, lax.reduce_sum(dot_v, axes=(0,)), dacc)
                naacc = jnp.where(sel, lax.reduce_sum(na_v, axes=(0,)), naacc)
                nbacc = jnp.where(sel, lax.reduce_sum(nb_v, axes=(0,)), nbacc)
            logit = dacc * scale * _rsqrt(naacc * nbacc)
            m_old = m_ref[...]
            m_new = jnp.maximum(m_old, logit)
            s_ref[...] = s_ref[...] * jnp.exp(m_old - m_new) + jnp.exp(logit - m_new)
            m_ref[...] = m_new
            t_ref[...] = t_ref[...] + logit * posf
            return 0

        lax.fori_loop(0, G, grp, 0)

    for b in range(NBUF):
        issue(b, b)

    def outer(k, _):
        gbase = k * NBUF
        for b in range(NBUF):
            j = gbase + b
            drain(b)
            compute(b)
            pl.when(j + NBUF < NCH)(functools.partial(issue, j + NBUF, b))
        return 0

    lax.fori_loop(0, NCH // NBUF, outer, 0)

    stage_v[pl.ds(0, L)] = m_ref[...]
    stage_v[pl.ds(L, L)] = s_ref[...]
    stage_v[pl.ds(2 * L, L)] = t_ref[...]
    pltpu.sync_copy(stage_v, out.at[wid])


_sc_stage = pl.kernel(
    _sc_body,
    out_type=jax.ShapeDtypeStruct((NW, 3 * L), f32),
    mesh=plsc.VectorSubcoreMesh(core_axis_name="c", subcore_axis_name="s"),
    compiler_params=pltpu.CompilerParams(
        needs_layout_passes=False,
        use_tc_tiling_on_sc=False,
    ),
    scratch_types=[
        pltpu.VMEM((PW,), i32),
        pltpu.VMEM((PW,), i32),
        pltpu.VMEM((CH, D), f32),
        pltpu.VMEM((CH, D), f32),
        pltpu.VMEM((CH, D), f32),
        pltpu.VMEM((CH, D), f32),
        pltpu.VMEM((CH, D), f32),
        pltpu.VMEM((CH, D), f32),
        pltpu.VMEM((CH, D), f32),
        pltpu.VMEM((CH, D), f32),
        pltpu.VMEM((L,), f32),
        pltpu.VMEM((3 * L,), f32),
        pltpu.VMEM((L,), f32),
        pltpu.VMEM((L,), f32),
        pltpu.VMEM((L,), f32),
        pltpu.SemaphoreType.DMA,
        pltpu.SemaphoreType.DMA,
        pltpu.SemaphoreType.DMA,
        pltpu.SemaphoreType.DMA,
    ],
)


def _combine_body(p_ref, o_ref):
    p = p_ref[...]
    m = p[:, 0:L]
    se = p[:, L:2 * L]
    t = p[:, 2 * L:3 * L]
    mx = jnp.max(m)
    ssum = jnp.sum(se * jnp.exp(m - mx))
    tsum = jnp.sum(t)
    o_ref[0, 0] = f32(P) * (mx + jnp.log(ssum)) - tsum


_combine = pl.pallas_call(
    _combine_body,
    out_shape=jax.ShapeDtypeStruct((1, 1), f32),
    out_specs=pl.BlockSpec(memory_space=pltpu.MemorySpace.SMEM),
)


def kernel(embeddings, scale, labels, anc1_indices, pos_indices, anc2_indices, neg_indices):
    del labels
    a1 = anc1_indices.astype(i32)
    po = pos_indices.astype(i32)
    a2 = anc2_indices.astype(i32)
    ne = neg_indices.astype(i32)
    scl = jnp.broadcast_to(scale.astype(f32), (L,))
    partials = _sc_stage(embeddings, a1, po, a2, ne, scl)
    return _combine(partials)[0, 0]


# speedup vs baseline: 1.0234x; 1.0234x over previous
import functools

import jax
import jax.numpy as jnp
from jax import lax
from jax.experimental import pallas as pl
from jax.experimental.pallas import tpu as pltpu
from jax.experimental.pallas import tpu_sc as plsc

P = 131072
D = 64
L = 16
NW = 32
PW = 2 * P // NW
CH = 64
NCH = PW // CH
NBUF = 4
G = CH // L

f32 = jnp.float32
i32 = jnp.int32


def _rsqrt(x):
    i = lax.bitcast_convert_type(x, i32)
    y = lax.bitcast_convert_type(jnp.int32(0x5F3759DF) - jnp.right_shift(i, 1), f32)
    for _ in range(3):
        y = y * (f32(1.5) - f32(0.5) * x * y * y)
    return y


def _sc_body(emb, a1, po, a2, ne, scl, out,
             aidx_v, bidx_v,
             ga0, ga1, ga2, ga3, gb0, gb1, gb2, gb3,
             ar0, ar1, ar2, ar3, br0, br1, br2, br3,
             scl_v, stage_v, m_ref, s_ref, t_ref,
             sem0, sem1, sem2, sem3):
    c = lax.axis_index("c")
    s = lax.axis_index("s")
    wid = s * 2 + c
    lane = lax.iota(i32, L)
    half = wid < (NW // 2)
    w2 = jnp.where(half, wid, wid - NW // 2)
    posf = jnp.where(half, f32(1.0), f32(0.0))
    base = w2 * PW

    pltpu.sync_copy(scl, scl_v)

    @pl.when(half)
    def _():
        pltpu.sync_copy(a1.at[pl.ds(base, PW)], aidx_v)
        pltpu.sync_copy(po.at[pl.ds(base, PW)], bidx_v)

    @pl.when(jnp.logical_not(half))
    def _():
        pltpu.sync_copy(a2.at[pl.ds(base, PW)], aidx_v)
        pltpu.sync_copy(ne.at[pl.ds(base, PW)], bidx_v)

    m_ref[...] = jnp.full((L,), -3.0e38, f32)
    s_ref[...] = jnp.zeros((L,), f32)
    t_ref[...] = jnp.zeros((L,), f32)

    gaidx = [ga0, ga1, ga2, ga3]
    gbidx = [gb0, gb1, gb2, gb3]
    arows = [ar0, ar1, ar2, ar3]
    brows = [br0, br1, br2, br3]
    sems = [sem0, sem1, sem2, sem3]

    def issue(j, b):
        for k in range(CH // L):
            sl = pl.ds(k * L, L)
            gaidx[b][sl] = jnp.right_shift(aidx_v[pl.ds(j * CH + k * L, L)], 1)
            gbidx[b][sl] = jnp.right_shift(bidx_v[pl.ds(j * CH + k * L, L)], 1)
        pltpu.async_copy(emb.at[gaidx[b]], arows[b], sems[b])
        pltpu.async_copy(emb.at[gbidx[b]], brows[b], sems[b])

    def drain(b):
        pltpu.make_async_copy(emb.at[gaidx[b]], arows[b], sems[b]).wait()
        pltpu.make_async_copy(emb.at[gbidx[b]], brows[b], sems[b]).wait()

    def compute(j, b):
        a_r, b_r = arows[b], brows[b]
        scale = scl_v[...]

        def grp(g, _):
            pbase = j * CH + g * L
            ha_vec = jnp.left_shift(jnp.bitwise_and(aidx_v[pl.ds(pbase, L)], 1), 6)
            hb_vec = jnp.left_shift(jnp.bitwise_and(bidx_v[pl.ds(pbase, L)], 1), 6)
            dacc = jnp.zeros((L,), f32)
            naacc = jnp.zeros((L,), f32)
            nbacc = jnp.zeros((L,), f32)
            for i in range(L):
                p = g * L + i
                ha = ha_vec[i]
                hb = hb_vec[i]
                dot_v = jnp.zeros((L,), f32)
                na_v = jnp.zeros((L,), f32)
                nb_v = jnp.zeros((L,), f32)
                for k in range(D // L):
                    av = a_r[p, pl.ds(ha + k * L, L)]
                    bv = b_r[p, pl.ds(hb + k * L, L)]
                    dot_v = dot_v + av * bv
                    na_v = na_v + av * av
                    nb_v = nb_v + bv * bv
                sel = lane == i
                dacc = jnp.where(sel, lax.reduce_sum(dot_v, axes=(0,)), dacc)
                naacc = jnp.where(sel, lax.reduce_sum(na_v, axes=(0,)), naacc)
                nbacc = jnp.where(sel, lax.reduce_sum(nb_v, axes=(0,)), nbacc)
            logit = dacc * scale * _rsqrt(naacc * nbacc)
            m_old = m_ref[...]
            m_new = jnp.maximum(m_old, logit)
            s_ref[...] = s_ref[...] * jnp.exp(m_old - m_new) + jnp.exp(logit - m_new)
            m_ref[...] = m_new
            t_ref[...] = t_ref[...] + logit * posf
            return 0

        lax.fori_loop(0, G, grp, 0)

    for b in range(NBUF):
        issue(b, b)

    def outer(k, _):
        gbase = k * NBUF
        for b in range(NBUF):
            j = gbase + b
            drain(b)
            compute(j, b)
            pl.when(j + NBUF < NCH)(functools.partial(issue, j + NBUF, b))
        return 0

    lax.fori_loop(0, NCH // NBUF, outer, 0)

    stage_v[pl.ds(0, L)] = m_ref[...]
    stage_v[pl.ds(L, L)] = s_ref[...]
    stage_v[pl.ds(2 * L, L)] = t_ref[...]
    pltpu.sync_copy(stage_v, out.at[wid])


_sc_stage = pl.kernel(
    _sc_body,
    out_type=jax.ShapeDtypeStruct((NW, 3 * L), f32),
    mesh=plsc.VectorSubcoreMesh(core_axis_name="c", subcore_axis_name="s"),
    compiler_params=pltpu.CompilerParams(
        needs_layout_passes=False,
        use_tc_tiling_on_sc=False,
    ),
    scratch_types=[
        pltpu.VMEM((PW,), i32),
        pltpu.VMEM((PW,), i32),
        pltpu.VMEM((CH,), i32),
        pltpu.VMEM((CH,), i32),
        pltpu.VMEM((CH,), i32),
        pltpu.VMEM((CH,), i32),
        pltpu.VMEM((CH,), i32),
        pltpu.VMEM((CH,), i32),
        pltpu.VMEM((CH,), i32),
        pltpu.VMEM((CH,), i32),
        pltpu.VMEM((CH, 2 * D), f32),
        pltpu.VMEM((CH, 2 * D), f32),
        pltpu.VMEM((CH, 2 * D), f32),
        pltpu.VMEM((CH, 2 * D), f32),
        pltpu.VMEM((CH, 2 * D), f32),
        pltpu.VMEM((CH, 2 * D), f32),
        pltpu.VMEM((CH, 2 * D), f32),
        pltpu.VMEM((CH, 2 * D), f32),
        pltpu.VMEM((L,), f32),
        pltpu.VMEM((3 * L,), f32),
        pltpu.VMEM((L,), f32),
        pltpu.VMEM((L,), f32),
        pltpu.VMEM((L,), f32),
        pltpu.SemaphoreType.DMA,
        pltpu.SemaphoreType.DMA,
        pltpu.SemaphoreType.DMA,
        pltpu.SemaphoreType.DMA,
    ],
)


def _combine_body(p_ref, o_ref):
    p = p_ref[...]
    m = p[:, 0:L]
    se = p[:, L:2 * L]
    t = p[:, 2 * L:3 * L]
    mx = jnp.max(m)
    ssum = jnp.sum(se * jnp.exp(m - mx))
    tsum = jnp.sum(t)
    o_ref[0, 0] = f32(P) * (mx + jnp.log(ssum)) - tsum


_combine = pl.pallas_call(
    _combine_body,
    out_shape=jax.ShapeDtypeStruct((1, 1), f32),
    out_specs=pl.BlockSpec(memory_space=pltpu.MemorySpace.SMEM),
)


def kernel(embeddings, scale, labels, anc1_indices, pos_indices, anc2_indices, neg_indices):
    del labels
    emb2 = embeddings.reshape(500000, 128)
    a1 = anc1_indices.astype(i32)
    po = pos_indices.astype(i32)
    a2 = anc2_indices.astype(i32)
    ne = neg_indices.astype(i32)
    scl = jnp.broadcast_to(scale.astype(f32), (L,))
    partials = _sc_stage(emb2, a1, po, a2, ne, scl)
    return _combine(partials)[0, 0]
